# baseline (speedup 1.0000x reference)
import functools

import jax
import jax.numpy as jnp
from jax import lax
from jax.experimental import pallas as pl
from jax.experimental.pallas import tpu as pltpu
from jax.experimental.pallas import tpu_sc as plsc

N = 50000
G = 256
NC = 2
NS = 16
NW = NC * NS
NPAD = 50176
RB = 512
NRB = NPAD // RB
SBK = 1792

_MESH = dict(mesh=plsc.VectorSubcoreMesh(core_axis_name="c", subcore_axis_name="s"))
_NOLAYOUT = pltpu.CompilerParams(needs_layout_passes=False)


def _make_deg(ept):
    nblk = ept // SBK
    sl = NPAD // NS

    def body(dst_hbm, deg_hbm, dst_st, didx, dval, zbuf, acc, sem):
        c = lax.axis_index("c")
        s = lax.axis_index("s")
        wid = c * NS + s

        def z(i, _):
            zbuf[pl.ds(i * 16, 16)] = jnp.zeros((16,), jnp.float32)
            return 0

        lax.fori_loop(0, sl // 16, z, 0)
        pltpu.sync_copy(zbuf, acc.at[pl.ds(s * sl, sl)])
        plsc.subcore_barrier()

        def blk(h, _):
            base = wid * ept + h * SBK
            pltpu.sync_copy(dst_hbm.at[pl.ds(base, SBK)], dst_st)

            def chunk(j, _):
                for t in range(8):
                    dv = dst_st[pl.ds(j * 128 + t * 16, 16)]
                    m = dv >= 0
                    didx[pl.ds(t * 16, 16)] = jnp.where(m, dv, 0)
                    dval[pl.ds(t * 16, 16)] = jnp.where(m, 1.0, 0.0)
                pltpu.sync_copy(dval, acc.at[didx], add=True)
                return 0

            lax.fori_loop(0, SBK // 128, chunk, 0)
            return 0

        lax.fori_loop(0, nblk, blk, 0)
        plsc.subcore_barrier()
        pltpu.sync_copy(acc.at[pl.ds(s * sl, sl)], zbuf)
        pltpu.sync_copy(zbuf, deg_hbm.at[pl.ds(c * NPAD + s * sl, sl)])

    return pl.kernel(
        body,
        out_type=jax.ShapeDtypeStruct((NC * NPAD,), jnp.float32),
        scratch_types=[
            pltpu.VMEM((SBK,), jnp.int32),
            pltpu.VMEM((128,), jnp.int32),
            pltpu.VMEM((128,), jnp.float32),
            pltpu.VMEM((sl,), jnp.float32),
            pltpu.VMEM_SHARED((NPAD,), jnp.float32),
            pltpu.SemaphoreType.DMA,
        ],
        **_MESH,
    )


def _make_agg(npass, ept, expand):
    C = NPAD // npass
    B = 64
    PB = B * expand
    nblk = ept // SBK
    AR = expand * C
    zr = AR // NS
    WBK = 8

    def body(src_hbm, dst_hbm, u_hbm, y_hbm,
             src_st, dst_st, sel, gidx, gdl, gbuf, acc, sem):
        c = lax.axis_index("c")
        s = lax.axis_index("s")
        wid = c * NS + s
        lane = lax.iota(jnp.int32, 16)

        def one_pass(p, _):
            lo = p * C
            lop = p * AR

            def zrow(i, _):
                for j in range(8):
                    gbuf[i, pl.ds(j * 16, 16)] = jnp.zeros((16,), jnp.float32)
                return 0

            lax.fori_loop(0, WBK, zrow, 0)

            def zacc(k, _):
                pltpu.sync_copy(gbuf.at[pl.ds(0, WBK), :],
                                acc.at[pl.ds(s * zr + k * WBK, WBK), :])
                return 0

            lax.fori_loop(0, zr // WBK, zacc, 0)
            plsc.subcore_barrier()

            def blk(h, _):
                base = wid * ept + h * SBK
                pltpu.sync_copy(src_hbm.at[pl.ds(base, SBK)], src_st)
                pltpu.sync_copy(dst_hbm.at[pl.ds(base, SBK)], dst_st)

                def vrg(i, lanecnt):
                    sv = src_st[pl.ds(i * 16, 16)]
                    dv = dst_st[pl.ds(i * 16, 16)]
                    m = (dv >= lo) & (dv < lo + C)
                    packed = jnp.bitwise_or(jnp.bitwise_and(sv, 0xFFFF),
                                            jnp.left_shift(dv - lo, 16))
                    plsc.store_scatter(sel, [lanecnt * 16 + lane], packed, mask=m)
                    return lanecnt + m.astype(jnp.int32)

                lanecnt = lax.fori_loop(0, SBK // 16, vrg,
                                        jnp.zeros((16,), jnp.int32))

                maxv = jnp.int32(0)
                for bit in (64, 32, 16, 8, 4, 2, 1):
                    cand = maxv + bit
                    anyge = plsc.all_reduce_population_count(lanecnt >= cand)[0] > 0
                    maxv = jnp.where(anyge, cand, maxv)

                nch = (maxv + 3) // 4

                def chunk(j, _):
                    for rr in range(4):
                        r = j * 4 + rr
                        valid = r < lanecnt
                        v = sel[pl.ds(r * 16, 16)]
                        sv = jnp.bitwise_and(v, 0xFFFF)
                        dl = lax.shift_right_logical(v, 16)
                        sv = jnp.where(valid, sv, 0)
                        dl = jnp.where(valid, dl, C)
                        if expand == 1:
                            gidx[pl.ds(rr * 16, 16)] = sv
                            gdl[pl.ds(rr * 16, 16)] = dl
                        else:
                            sv2 = sv * 2
                            dl2 = dl * 2
                            pos = rr * 32 + lane * 2
                            plsc.store_scatter(gidx, [pos], sv2)
                            plsc.store_scatter(gidx, [pos + 1], sv2 + 1)
                            plsc.store_scatter(gdl, [pos], dl2)
                            plsc.store_scatter(gdl, [pos + 1], dl2 + 1)
                    pltpu.async_copy(u_hbm.at[gidx], gbuf, sem).wait()
                    pltpu.sync_copy(gbuf, acc.at[gdl], add=True)
                    return 0

                lax.fori_loop(0, nch, chunk, 0)
                return 0

            lax.fori_loop(0, nblk, blk, 0)
            plsc.subcore_barrier()

            def wb(k, _):
                r = s * zr + k * WBK
                pltpu.sync_copy(acc.at[pl.ds(r, WBK), :], gbuf.at[pl.ds(0, WBK), :])
                pltpu.sync_copy(gbuf.at[pl.ds(0, WBK), :],
                                y_hbm.at[c, pl.ds(lop + r, WBK), :])
                return 0

            lax.fori_loop(0, zr // WBK, wb, 0)
            plsc.subcore_barrier()
            return 0

        lax.fori_loop(0, npass, one_pass, 0)

    return pl.kernel(
        body,
        out_type=jax.ShapeDtypeStruct((NC, NPAD * expand, 128), jnp.float32),
        scratch_types=[
            pltpu.VMEM((SBK,), jnp.int32),
            pltpu.VMEM((SBK,), jnp.int32),
            pltpu.VMEM((SBK,), jnp.int32),
            pltpu.VMEM((PB,), jnp.int32),
            pltpu.VMEM((PB,), jnp.int32),
            pltpu.VMEM((PB, 128), jnp.float32),
            pltpu.VMEM_SHARED((AR + 8, 128), jnp.float32),
            pltpu.SemaphoreType.DMA,
        ],
        compiler_params=_NOLAYOUT,
        **_MESH,
    )


def _make_pool():
    rows = NPAD // NW
    BRK = 56

    def body(h_hbm, b_hbm, out_hbm, hbuf, bbuf, macc, sem):
        c = lax.axis_index("c")
        s = lax.axis_index("s")
        wid = c * NS + s
        base = wid * rows

        def im(r, _):
            for j in range(16):
                macc[r, pl.ds(j * 16, 16)] = jnp.full((16,), -jnp.inf, jnp.float32)
            return 0

        lax.fori_loop(0, G, im, 0)
        pltpu.sync_copy(b_hbm.at[pl.ds(base, rows)], bbuf.at[pl.ds(0, rows)])

        def blk(k, _):
            pltpu.sync_copy(h_hbm.at[pl.ds(base + k * BRK, BRK), :], hbuf)

            def row(i, _):
                gr = base + k * BRK + i

                @pl.when(gr < N)
                def _():
                    g = bbuf[pl.ds(k * BRK + i, 16)][0]
                    for j in range(16):
                        a = macc[g, pl.ds(j * 16, 16)]
                        hv = hbuf[i, pl.ds(j * 16, 16)]
                        macc[g, pl.ds(j * 16, 16)] = jnp.maximum(a, hv)

                return 0

            lax.fori_loop(0, BRK, row, 0)
            return 0

        lax.fori_loop(0, rows // BRK, blk, 0)
        pltpu.sync_copy(macc, out_hbm.at[wid])

    return pl.kernel(
        body,
        out_type=jax.ShapeDtypeStruct((NW, G, 256), jnp.float32),
        scratch_types=[
            pltpu.VMEM((BRK, 256), jnp.float32),
            pltpu.VMEM((rows + 16,), jnp.int32),
            pltpu.VMEM((G, 256), jnp.float32),
            pltpu.SemaphoreType.DMA,
        ],
        **_MESH,
    )


def _prep_body(deg0_ref, deg1_ref, x_ref, w_ref, t1_ref, dinv_ref):
    deg = deg0_ref[...] + deg1_ref[...] + 1.0
    d = lax.rsqrt(deg)
    dinv_ref[...] = d
    t1_ref[...] = jnp.dot(x_ref[...], w_ref[...],
                          preferred_element_type=jnp.float32) * d


def _prep(deg0, deg1, xpad, W1p):
    return pl.pallas_call(
        _prep_body,
        grid=(NRB,),
        in_specs=[
            pl.BlockSpec((RB, 1), lambda i: (i, 0)),
            pl.BlockSpec((RB, 1), lambda i: (i, 0)),
            pl.BlockSpec((RB, 16), lambda i: (i, 0)),
            pl.BlockSpec((16, 128), lambda i: (0, 0)),
        ],
        out_specs=[
            pl.BlockSpec((RB, 128), lambda i: (i, 0)),
            pl.BlockSpec((RB, 1), lambda i: (i, 0)),
        ],
        out_shape=[
            jax.ShapeDtypeStruct((NPAD, 128), jnp.float32),
            jax.ShapeDtypeStruct((NPAD, 1), jnp.float32),
        ],
    )(deg0, deg1, xpad, W1p)


def _elem_body(y_ref, t_ref, dinv_ref, b_ref, out_ref):
    d = dinv_ref[...]
    h = jnp.maximum((y_ref[0] + y_ref[1] + t_ref[...]) * d + b_ref[...], 0.0)
    out_ref[...] = h * d


def _elem(y, t, dinv, b):
    w = t.shape[1]
    return pl.pallas_call(
        _elem_body,
        grid=(NRB,),
        in_specs=[
            pl.BlockSpec((2, RB, w), lambda i: (0, i, 0)),
            pl.BlockSpec((RB, w), lambda i: (i, 0)),
            pl.BlockSpec((RB, 1), lambda i: (i, 0)),
            pl.BlockSpec((1, w), lambda i: (0, 0)),
        ],
        out_specs=pl.BlockSpec((RB, w), lambda i: (i, 0)),
        out_shape=jax.ShapeDtypeStruct((NPAD, w), jnp.float32),
    )(y, t, dinv, b)


def _layer_body(y_ref, t_ref, dinv_ref, w_ref, b_ref, out_ref, *, relu, scale_out):
    d = dinv_ref[...]
    z = (y_ref[0] + y_ref[1] + t_ref[...]) * d
    o = jnp.dot(z, w_ref[...], preferred_element_type=jnp.float32) + b_ref[...]
    if relu:
        o = jnp.maximum(o, 0.0)
    if scale_out:
        o = o * d
    out_ref[...] = o


def _layer(y, t, dinv, w, b, *, relu, scale_out):
    wi, wo = w.shape
    return pl.pallas_call(
        functools.partial(_layer_body, relu=relu, scale_out=scale_out),
        grid=(NRB,),
        in_specs=[
            pl.BlockSpec((2, RB, wi), lambda i: (0, i, 0)),
            pl.BlockSpec((RB, wi), lambda i: (i, 0)),
            pl.BlockSpec((RB, 1), lambda i: (i, 0)),
            pl.BlockSpec((wi, wo), lambda i: (0, 0)),
            pl.BlockSpec((1, wo), lambda i: (0, 0)),
        ],
        out_specs=pl.BlockSpec((RB, wo), lambda i: (i, 0)),
        out_shape=jax.ShapeDtypeStruct((NPAD, wo), jnp.float32),
    )(y, t, dinv, w, b)


def _head_body(p_ref, w2_ref, b2_ref, w3_ref, b3_ref, wl_ref, bl_ref, out_ref):
    hp = jnp.max(p_ref[...], axis=0)
    h = jax.nn.relu(jnp.dot(hp, w2_ref[...], preferred_element_type=jnp.float32) + b2_ref[...])
    h = jax.nn.relu(jnp.dot(h, w3_ref[...], preferred_element_type=jnp.float32) + b3_ref[...])
    out_ref[...] = jnp.dot(h, wl_ref[...], preferred_element_type=jnp.float32) + bl_ref[...]


def _head(pool, Wl2, bl2, Wl3, bl3, Wlin, blin):
    return pl.pallas_call(
        _head_body,
        out_shape=jax.ShapeDtypeStruct((G, Wlin.shape[1]), jnp.float32),
    )(pool, Wl2, bl2[None, :], Wl3, bl3[None, :], Wlin, blin[None, :])


def kernel(x, edge_index, batch, W1, b1, W2, b2, W3, b3, W4, b4, Wl2, bl2, Wl3, bl3, Wlin, blin):
    E = edge_index.shape[1]
    ept = ((E + NW * SBK - 1) // (NW * SBK)) * SBK
    epad = NW * ept
    src_p = jnp.concatenate([edge_index[0], jnp.zeros((epad - E,), jnp.int32)])
    dst_p = jnp.concatenate([edge_index[1], jnp.full((epad - E,), -1, jnp.int32)])
    xpad = jnp.pad(x, ((0, NPAD - N), (0, 16 - x.shape[1])))
    batch_p = jnp.pad(batch, (0, NPAD - N))
    W1p = jnp.pad(W1, ((0, 16 - W1.shape[0]), (0, 0)))

    degp = _make_deg(ept)(dst_p)
    degc = degp.reshape(NC, NPAD, 1)
    t1, dinv = _prep(degc[0], degc[1], xpad, W1p)

    agg128 = _make_agg(4, ept, 1)
    agg256 = _make_agg(8, ept, 2)

    def agg_w(agg, t, expand):
        y = agg(src_p, dst_p, t.reshape(NPAD * expand, 128))
        return y.reshape(NC, NPAD, 128 * expand)

    a1 = agg_w(agg128, t1, 1)
    t2 = _elem(a1, t1, dinv, b1[None, :])
    a2 = agg_w(agg128, t2, 1)
    t3 = _layer(a2, t2, dinv, W2, b2[None, :], relu=True, scale_out=True)
    a3 = agg_w(agg256, t3, 2)
    t4 = _layer(a3, t3, dinv, W3, b3[None, :], relu=True, scale_out=True)
    a4 = agg_w(agg256, t4, 2)
    h4 = _layer(a4, t4, dinv, W4, b4[None, :], relu=False, scale_out=False)

    pool = _make_pool()(h4, batch_p)
    out = _head(pool, Wl2, bl2, Wl3, bl3, Wlin, blin)
    return jnp.squeeze(out)


# speedup vs baseline: 8.5171x; 8.5171x over previous
import functools

import jax
import jax.numpy as jnp
from jax import lax
from jax.experimental import pallas as pl
from jax.experimental.pallas import tpu as pltpu
from jax.experimental.pallas import tpu_sc as plsc

N = 50000
G = 256
NC = 2
NS = 16
NW = NC * NS
NPAD = 50176
RB = 512
NRB = NPAD // RB
SBK = 3584

_MESH = dict(mesh=plsc.VectorSubcoreMesh(core_axis_name="c", subcore_axis_name="s"))
_NOLAYOUT = pltpu.CompilerParams(needs_layout_passes=False)


def _make_deg(ept):
    nblk = ept // SBK
    sl = NPAD // NS

    def body(dst_hbm, deg_hbm, dst_st, didx, dval, zbuf, acc, sem):
        c = lax.axis_index("c")
        s = lax.axis_index("s")
        wid = c * NS + s

        def z(i, _):
            zbuf[pl.ds(i * 16, 16)] = jnp.zeros((16,), jnp.float32)
            return 0

        lax.fori_loop(0, sl // 16, z, 0)
        pltpu.sync_copy(zbuf, acc.at[pl.ds(s * sl, sl)])
        plsc.subcore_barrier()

        def blk(h, _):
            base = wid * ept + h * SBK
            pltpu.sync_copy(dst_hbm.at[pl.ds(base, SBK)], dst_st)

            def chunk(j, _):
                for t in range(8):
                    dv = dst_st[pl.ds(j * 128 + t * 16, 16)]
                    m = dv >= 0
                    didx[pl.ds(t * 16, 16)] = jnp.where(m, dv, 0)
                    dval[pl.ds(t * 16, 16)] = jnp.where(m, 1.0, 0.0)
                pltpu.sync_copy(dval, acc.at[didx], add=True)
                return 0

            lax.fori_loop(0, SBK // 128, chunk, 0)
            return 0

        lax.fori_loop(0, nblk, blk, 0)
        plsc.subcore_barrier()
        pltpu.sync_copy(acc.at[pl.ds(s * sl, sl)], zbuf)
        pltpu.sync_copy(zbuf, deg_hbm.at[pl.ds(c * NPAD + s * sl, sl)])

    return pl.kernel(
        body,
        out_type=jax.ShapeDtypeStruct((NC * NPAD,), jnp.float32),
        scratch_types=[
            pltpu.VMEM((SBK,), jnp.int32),
            pltpu.VMEM((128,), jnp.int32),
            pltpu.VMEM((128,), jnp.float32),
            pltpu.VMEM((sl,), jnp.float32),
            pltpu.VMEM_SHARED((NPAD,), jnp.float32),
            pltpu.SemaphoreType.DMA,
        ],
        **_MESH,
    )


def _make_agg(npass, ept, expand, B):
    C = NPAD // npass
    NV = B // 16
    PB = B * expand
    AR = expand * C
    nblk = ept // SBK
    zr = AR // NS
    WBK = 8
    QCAP = SBK + B

    def body(src_hbm, dst_hbm, u_hbm, y_hbm,
             src_st, dst_st, sel, gidx, gdl, gbuf, acc, semA, semB):
        c = lax.axis_index("c")
        s = lax.axis_index("s")
        wid = c * NS + s
        lane = lax.iota(jnp.int32, 16)

        def views(slot):
            gi = gidx.at[pl.ds(slot * PB, PB)]
            gb = gbuf.at[pl.ds(slot * PB, PB), :]
            sem = semA if slot == 0 else semB
            return gi, gb, sem

        def build_fire(f, slot):
            for rr in range(NV):
                v = sel[pl.ds(f * B + rr * 16, 16)]
                sv = jnp.bitwise_and(v, 0xFFFF)
                dl = lax.shift_right_logical(v, 16)
                if expand == 1:
                    gidx[pl.ds(slot * PB + rr * 16, 16)] = sv
                    gdl[pl.ds(slot * PB + rr * 16, 16)] = dl
                else:
                    sv2 = sv * 2
                    dl2 = dl * 2
                    pos = slot * PB + rr * 32 + lane * 2
                    plsc.store_scatter(gidx, [pos], sv2)
                    plsc.store_scatter(gidx, [pos + 1], sv2 + 1)
                    plsc.store_scatter(gdl, [pos], dl2)
                    plsc.store_scatter(gdl, [pos + 1], dl2 + 1)
            gi, gb, sem = views(slot)
            pltpu.async_copy(u_hbm.at[gi], gb, sem)

        def retire(slot):
            gi, gb, sem = views(slot)
            pltpu.make_async_copy(u_hbm.at[gi], gb, sem).wait()
            pltpu.sync_copy(gb, acc.at[gdl.at[pl.ds(slot * PB, PB)]], add=True)

        def run_chunks(nfull):
            @pl.when(nfull > 0)
            def _():
                build_fire(0, 0)

            def pipe(f, _):
                odd = lax.rem(f, 2) == 1

                @pl.when(odd)
                def _():
                    build_fire(f, 1)
                    retire(0)

                @pl.when(jnp.logical_not(odd))
                def _():
                    build_fire(f, 0)
                    retire(1)

                return 0

            lax.fori_loop(1, nfull, pipe, 0)

            @pl.when(nfull > 0)
            def _():
                last_odd = lax.rem(nfull - 1, 2) == 1

                @pl.when(last_odd)
                def _():
                    retire(1)

                @pl.when(jnp.logical_not(last_odd))
                def _():
                    retire(0)

        def one_pass(p, _):
            lo = p * C

            def zrow(i, _):
                for j in range(8):
                    gbuf[i, pl.ds(j * 16, 16)] = jnp.zeros((16,), jnp.float32)
                return 0

            lax.fori_loop(0, WBK, zrow, 0)

            def zacc(k, _):
                pltpu.sync_copy(gbuf.at[pl.ds(0, WBK), :],
                                acc.at[pl.ds(s * zr + k * WBK, WBK), :])
                return 0

            lax.fori_loop(0, zr // WBK, zacc, 0)
            plsc.subcore_barrier()

            def blk(h, qv):
                base = wid * ept + h * SBK
                pltpu.sync_copy(src_hbm.at[pl.ds(base, SBK)], src_st)
                pltpu.sync_copy(dst_hbm.at[pl.ds(base, SBK)], dst_st)

                def vrg(i, q):
                    sv = src_st[pl.ds(i * 16, 16)]
                    dv = dst_st[pl.ds(i * 16, 16)]
                    m = (dv >= lo) & (dv < lo + C)
                    mi = jnp.where(m, 1, 0)
                    pos = q + plsc.cumsum(mi) - mi
                    packed = jnp.bitwise_or(jnp.bitwise_and(sv, 0xFFFF),
                                            jnp.left_shift(dv - lo, 16))
                    plsc.store_scatter(sel, [pos], packed, mask=m)
                    return q + plsc.all_reduce_population_count(m)

                qv = lax.fori_loop(0, SBK // 16, vrg, qv)
                qlen = qv[0]
                nfull = qlen // B
                run_chunks(nfull)
                for t in range(NV):
                    sel[pl.ds(t * 16, 16)] = sel[pl.ds(nfull * B + t * 16, 16)]
                return qv - nfull * B

            qv = lax.fori_loop(0, nblk, blk, jnp.zeros((16,), jnp.int32))

            @pl.when(qv[0] > 0)
            def _():
                dmy = jnp.full((16,), C << 16, jnp.int32)
                for t in range(NV):
                    pos = t * 16 + lane
                    plsc.store_scatter(sel, [pos], dmy, mask=pos >= qv)
                build_fire(0, 0)
                retire(0)

            plsc.subcore_barrier()

            def wb(k, _):
                r = s * zr + k * WBK
                pltpu.sync_copy(acc.at[pl.ds(r, WBK), :], gbuf.at[pl.ds(0, WBK), :])
                pltpu.sync_copy(gbuf.at[pl.ds(0, WBK), :],
                                y_hbm.at[c, pl.ds(p * AR + r, WBK), :])
                return 0

            lax.fori_loop(0, zr // WBK, wb, 0)
            plsc.subcore_barrier()
            return 0

        lax.fori_loop(0, npass, one_pass, 0)

    return pl.kernel(
        body,
        out_type=jax.ShapeDtypeStruct((NC, NPAD * expand, 128), jnp.float32),
        scratch_types=[
            pltpu.VMEM((SBK,), jnp.int32),
            pltpu.VMEM((SBK,), jnp.int32),
            pltpu.VMEM((QCAP,), jnp.int32),
            pltpu.VMEM((2 * PB,), jnp.int32),
            pltpu.VMEM((2 * PB,), jnp.int32),
            pltpu.VMEM((2 * PB, 128), jnp.float32),
            pltpu.VMEM_SHARED((AR + 8, 128), jnp.float32),
            pltpu.SemaphoreType.DMA,
            pltpu.SemaphoreType.DMA,
        ],
        compiler_params=_NOLAYOUT,
        **_MESH,
    )


def _make_pool():
    rows = NPAD // NW
    BRK = 56

    def body(h_hbm, b_hbm, out_hbm, hbuf, bbuf, macc, sem):
        c = lax.axis_index("c")
        s = lax.axis_index("s")
        wid = c * NS + s
        base = wid * rows

        def im(r, _):
            for j in range(16):
                macc[r, pl.ds(j * 16, 16)] = jnp.full((16,), -jnp.inf, jnp.float32)
            return 0

        lax.fori_loop(0, G, im, 0)
        pltpu.sync_copy(b_hbm.at[pl.ds(base, rows)], bbuf.at[pl.ds(0, rows)])

        def blk(k, _):
            pltpu.sync_copy(h_hbm.at[pl.ds(base + k * BRK, BRK), :], hbuf)

            def row(i, _):
                gr = base + k * BRK + i

                @pl.when(gr < N)
                def _():
                    g = bbuf[pl.ds(k * BRK + i, 16)][0]
                    for j in range(16):
                        a = macc[g, pl.ds(j * 16, 16)]
                        hv = hbuf[i, pl.ds(j * 16, 16)]
                        macc[g, pl.ds(j * 16, 16)] = jnp.maximum(a, hv)

                return 0

            lax.fori_loop(0, BRK, row, 0)
            return 0

        lax.fori_loop(0, rows // BRK, blk, 0)
        pltpu.sync_copy(macc, out_hbm.at[wid])

    return pl.kernel(
        body,
        out_type=jax.ShapeDtypeStruct((NW, G, 256), jnp.float32),
        scratch_types=[
            pltpu.VMEM((BRK, 256), jnp.float32),
            pltpu.VMEM((rows + 16,), jnp.int32),
            pltpu.VMEM((G, 256), jnp.float32),
            pltpu.SemaphoreType.DMA,
        ],
        **_MESH,
    )


def _prep_body(deg0_ref, deg1_ref, x_ref, w_ref, t1_ref, dinv_ref):
    deg = deg0_ref[...] + deg1_ref[...] + 1.0
    d = lax.rsqrt(deg)
    dinv_ref[...] = d
    t1_ref[...] = jnp.dot(x_ref[...], w_ref[...],
                          preferred_element_type=jnp.float32) * d


def _prep(deg0, deg1, xpad, W1p):
    return pl.pallas_call(
        _prep_body,
        grid=(NRB,),
        in_specs=[
            pl.BlockSpec((RB, 1), lambda i: (i, 0)),
            pl.BlockSpec((RB, 1), lambda i: (i, 0)),
            pl.BlockSpec((RB, 16), lambda i: (i, 0)),
            pl.BlockSpec((16, 128), lambda i: (0, 0)),
        ],
        out_specs=[
            pl.BlockSpec((RB, 128), lambda i: (i, 0)),
            pl.BlockSpec((RB, 1), lambda i: (i, 0)),
        ],
        out_shape=[
            jax.ShapeDtypeStruct((NPAD, 128), jnp.float32),
            jax.ShapeDtypeStruct((NPAD, 1), jnp.float32),
        ],
    )(deg0, deg1, xpad, W1p)


def _elem_body(y_ref, t_ref, dinv_ref, b_ref, out_ref):
    d = dinv_ref[...]
    h = jnp.maximum((y_ref[0] + y_ref[1] + t_ref[...]) * d + b_ref[...], 0.0)
    out_ref[...] = h * d


def _elem(y, t, dinv, b):
    w = t.shape[1]
    return pl.pallas_call(
        _elem_body,
        grid=(NRB,),
        in_specs=[
            pl.BlockSpec((2, RB, w), lambda i: (0, i, 0)),
            pl.BlockSpec((RB, w), lambda i: (i, 0)),
            pl.BlockSpec((RB, 1), lambda i: (i, 0)),
            pl.BlockSpec((1, w), lambda i: (0, 0)),
        ],
        out_specs=pl.BlockSpec((RB, w), lambda i: (i, 0)),
        out_shape=jax.ShapeDtypeStruct((NPAD, w), jnp.float32),
    )(y, t, dinv, b)


def _layer_body(y_ref, t_ref, dinv_ref, w_ref, b_ref, out_ref, *, relu, scale_out):
    d = dinv_ref[...]
    z = (y_ref[0] + y_ref[1] + t_ref[...]) * d
    o = jnp.dot(z, w_ref[...], preferred_element_type=jnp.float32) + b_ref[...]
    if relu:
        o = jnp.maximum(o, 0.0)
    if scale_out:
        o = o * d
    out_ref[...] = o


def _layer(y, t, dinv, w, b, *, relu, scale_out):
    wi, wo = w.shape
    return pl.pallas_call(
        functools.partial(_layer_body, relu=relu, scale_out=scale_out),
        grid=(NRB,),
        in_specs=[
            pl.BlockSpec((2, RB, wi), lambda i: (0, i, 0)),
            pl.BlockSpec((RB, wi), lambda i: (i, 0)),
            pl.BlockSpec((RB, 1), lambda i: (i, 0)),
            pl.BlockSpec((wi, wo), lambda i: (0, 0)),
            pl.BlockSpec((1, wo), lambda i: (0, 0)),
        ],
        out_specs=pl.BlockSpec((RB, wo), lambda i: (i, 0)),
        out_shape=jax.ShapeDtypeStruct((NPAD, wo), jnp.float32),
    )(y, t, dinv, w, b)


def _head_body(p_ref, w2_ref, b2_ref, w3_ref, b3_ref, wl_ref, bl_ref, out_ref):
    hp = jnp.max(p_ref[...], axis=0)
    h = jax.nn.relu(jnp.dot(hp, w2_ref[...], preferred_element_type=jnp.float32) + b2_ref[...])
    h = jax.nn.relu(jnp.dot(h, w3_ref[...], preferred_element_type=jnp.float32) + b3_ref[...])
    out_ref[...] = jnp.dot(h, wl_ref[...], preferred_element_type=jnp.float32) + bl_ref[...]


def _head(pool, Wl2, bl2, Wl3, bl3, Wlin, blin):
    return pl.pallas_call(
        _head_body,
        out_shape=jax.ShapeDtypeStruct((G, Wlin.shape[1]), jnp.float32),
    )(pool, Wl2, bl2[None, :], Wl3, bl3[None, :], Wlin, blin[None, :])


def kernel(x, edge_index, batch, W1, b1, W2, b2, W3, b3, W4, b4, Wl2, bl2, Wl3, bl3, Wlin, blin):
    E = edge_index.shape[1]
    ept = ((E + NW * SBK - 1) // (NW * SBK)) * SBK
    epad = NW * ept
    src_p = jnp.concatenate([edge_index[0], jnp.zeros((epad - E,), jnp.int32)])
    dst_p = jnp.concatenate([edge_index[1], jnp.full((epad - E,), -1, jnp.int32)])
    xpad = jnp.pad(x, ((0, NPAD - N), (0, 16 - x.shape[1])))
    batch_p = jnp.pad(batch, (0, NPAD - N))
    W1p = jnp.pad(W1, ((0, 16 - W1.shape[0]), (0, 0)))

    degp = _make_deg(ept)(dst_p)
    degc = degp.reshape(NC, NPAD, 1)
    t1, dinv = _prep(degc[0], degc[1], xpad, W1p)

    agg128 = _make_agg(7, ept, 1, 128)
    agg256 = _make_agg(14, ept, 2, 64)

    def agg_w(agg, t, expand):
        y = agg(src_p, dst_p, t.reshape(NPAD * expand, 128))
        return y.reshape(NC, NPAD, 128 * expand)

    a1 = agg_w(agg128, t1, 1)
    t2 = _elem(a1, t1, dinv, b1[None, :])
    a2 = agg_w(agg128, t2, 1)
    t3 = _layer(a2, t2, dinv, W2, b2[None, :], relu=True, scale_out=True)
    a3 = agg_w(agg256, t3, 2)
    t4 = _layer(a3, t3, dinv, W3, b3[None, :], relu=True, scale_out=True)
    a4 = agg_w(agg256, t4, 2)
    h4 = _layer(a4, t4, dinv, W4, b4[None, :], relu=False, scale_out=False)

    pool = _make_pool()(h4, batch_p)
    out = _head(pool, Wl2, bl2, Wl3, bl3, Wlin, blin)
    return jnp.squeeze(out)


# speedup vs baseline: 8.9722x; 1.0534x over previous
import functools

import jax
import jax.numpy as jnp
from jax import lax
from jax.experimental import pallas as pl
from jax.experimental.pallas import tpu as pltpu
from jax.experimental.pallas import tpu_sc as plsc

N = 50000
G = 256
NC = 2
NS = 16
NW = NC * NS
NPAD = 50176
RB = 512
NRB = NPAD // RB
SBK = 3584

_MESH = dict(mesh=plsc.VectorSubcoreMesh(core_axis_name="c", subcore_axis_name="s"))
_NOLAYOUT = pltpu.CompilerParams(needs_layout_passes=False)


def _make_deg(ept):
    nblk = ept // SBK
    sl = NPAD // NS

    def body(dst_hbm, deg_hbm, dst_st, didx, dval, zbuf, acc, sem):
        c = lax.axis_index("c")
        s = lax.axis_index("s")
        wid = c * NS + s

        def z(i, _):
            zbuf[pl.ds(i * 16, 16)] = jnp.zeros((16,), jnp.float32)
            return 0

        lax.fori_loop(0, sl // 16, z, 0)
        pltpu.sync_copy(zbuf, acc.at[pl.ds(s * sl, sl)])
        plsc.subcore_barrier()

        def blk(h, _):
            base = wid * ept + h * SBK
            pltpu.sync_copy(dst_hbm.at[pl.ds(base, SBK)], dst_st)

            def chunk(j, _):
                for t in range(8):
                    dv = dst_st[pl.ds(j * 128 + t * 16, 16)]
                    m = dv >= 0
                    didx[pl.ds(t * 16, 16)] = jnp.where(m, dv, 0)
                    dval[pl.ds(t * 16, 16)] = jnp.where(m, 1.0, 0.0)
                pltpu.sync_copy(dval, acc.at[didx], add=True)
                return 0

            lax.fori_loop(0, SBK // 128, chunk, 0)
            return 0

        lax.fori_loop(0, nblk, blk, 0)
        plsc.subcore_barrier()
        pltpu.sync_copy(acc.at[pl.ds(s * sl, sl)], zbuf)
        pltpu.sync_copy(zbuf, deg_hbm.at[pl.ds(c * NPAD + s * sl, sl)])

    return pl.kernel(
        body,
        out_type=jax.ShapeDtypeStruct((NC * NPAD,), jnp.float32),
        scratch_types=[
            pltpu.VMEM((SBK,), jnp.int32),
            pltpu.VMEM((128,), jnp.int32),
            pltpu.VMEM((128,), jnp.float32),
            pltpu.VMEM((sl,), jnp.float32),
            pltpu.VMEM_SHARED((NPAD,), jnp.float32),
            pltpu.SemaphoreType.DMA,
        ],
        **_MESH,
    )


def _make_agg(npass, ept, expand, B):
    C = NPAD // npass
    NV = B // 16
    PB = B * expand
    AR = expand * C
    nblk = ept // SBK
    zr = AR // NS
    WBK = 8
    QCAP = ept + B

    def body(src_hbm, dst_hbm, u_hbm, y_hbm,
             src_st, dst_st, sel, gidx, gdl, gbuf, acc, semA, semB):
        c = lax.axis_index("c")
        s = lax.axis_index("s")
        wid = c * NS + s
        lane = lax.iota(jnp.int32, 16)

        def views(slot):
            gi = gidx.at[pl.ds(slot * PB, PB)]
            gb = gbuf.at[pl.ds(slot * PB, PB), :]
            sem = semA if slot == 0 else semB
            return gi, gb, sem

        def build_fire(f, slot):
            for rr in range(NV):
                v = sel[pl.ds(f * B + rr * 16, 16)]
                sv = jnp.bitwise_and(v, 0xFFFF)
                dl = lax.shift_right_logical(v, 16)
                if expand == 1:
                    gidx[pl.ds(slot * PB + rr * 16, 16)] = sv
                    gdl[pl.ds(slot * PB + rr * 16, 16)] = dl
                else:
                    sv2 = sv * 2
                    dl2 = dl * 2
                    pos = slot * PB + rr * 32 + lane * 2
                    plsc.store_scatter(gidx, [pos], sv2)
                    plsc.store_scatter(gidx, [pos + 1], sv2 + 1)
                    plsc.store_scatter(gdl, [pos], dl2)
                    plsc.store_scatter(gdl, [pos + 1], dl2 + 1)
            gi, gb, sem = views(slot)
            pltpu.async_copy(u_hbm.at[gi], gb, sem)

        def retire(slot):
            gi, gb, sem = views(slot)
            pltpu.make_async_copy(u_hbm.at[gi], gb, sem).wait()
            pltpu.sync_copy(gb, acc.at[gdl.at[pl.ds(slot * PB, PB)]], add=True)

        def step(f, _):
            odd = lax.rem(f, 2) == 1

            @pl.when(odd)
            def _():
                build_fire(f, 1)

            @pl.when(jnp.logical_not(odd))
            def _():
                build_fire(f, 0)

            @pl.when((f >= 1) & odd)
            def _():
                retire(0)

            @pl.when((f >= 1) & jnp.logical_not(odd))
            def _():
                retire(1)

            return 0

        def one_pass(p, _):
            lo = p * C

            def zrow(i, _):
                for j in range(8):
                    gbuf[i, pl.ds(j * 16, 16)] = jnp.zeros((16,), jnp.float32)
                return 0

            lax.fori_loop(0, WBK, zrow, 0)

            def zacc(k, _):
                pltpu.sync_copy(gbuf.at[pl.ds(0, WBK), :],
                                acc.at[pl.ds(s * zr + k * WBK, WBK), :])
                return 0

            lax.fori_loop(0, zr // WBK, zacc, 0)
            plsc.subcore_barrier()

            def blk(h, carry):
                qv, nfdone = carry
                base = wid * ept + h * SBK
                pltpu.sync_copy(src_hbm.at[pl.ds(base, SBK)], src_st)
                pltpu.sync_copy(dst_hbm.at[pl.ds(base, SBK)], dst_st)

                def vrg(i, q):
                    sv = src_st[pl.ds(i * 16, 16)]
                    dv = dst_st[pl.ds(i * 16, 16)]
                    m = (dv >= lo) & (dv < lo + C)
                    mi = jnp.where(m, 1, 0)
                    pos = q + plsc.cumsum(mi) - mi
                    packed = jnp.bitwise_or(jnp.bitwise_and(sv, 0xFFFF),
                                            jnp.left_shift(dv - lo, 16))
                    plsc.store_scatter(sel, [pos], packed, mask=m)
                    return q + plsc.all_reduce_population_count(m)

                qv = lax.fori_loop(0, SBK // 16, vrg, qv)
                nfull = qv[0] // B
                lax.fori_loop(nfdone, nfull, step, 0)
                return qv, nfull

            qv, nfull = lax.fori_loop(0, nblk, blk,
                                      (jnp.zeros((16,), jnp.int32), jnp.int32(0)))

            qrem = qv[0] - nfull * B

            @pl.when(qrem > 0)
            def _():
                dmy = jnp.full((16,), C << 16, jnp.int32)
                for t in range(NV):
                    pos = nfull * B + t * 16 + lane
                    plsc.store_scatter(sel, [pos], dmy, mask=pos >= qv)
                step(nfull, 0)

            total = nfull + jnp.where(qrem > 0, 1, 0)
            last_odd = lax.rem(total - 1, 2) == 1

            @pl.when((total > 0) & last_odd)
            def _():
                retire(1)

            @pl.when((total > 0) & jnp.logical_not(last_odd))
            def _():
                retire(0)

            plsc.subcore_barrier()

            def wb(k, _):
                r = s * zr + k * WBK
                pltpu.sync_copy(acc.at[pl.ds(r, WBK), :], gbuf.at[pl.ds(0, WBK), :])
                pltpu.sync_copy(gbuf.at[pl.ds(0, WBK), :],
                                y_hbm.at[c, pl.ds(p * AR + r, WBK), :])
                return 0

            lax.fori_loop(0, zr // WBK, wb, 0)
            plsc.subcore_barrier()
            return 0

        lax.fori_loop(0, npass, one_pass, 0)

    return pl.kernel(
        body,
        out_type=jax.ShapeDtypeStruct((NC, NPAD * expand, 128), jnp.float32),
        scratch_types=[
            pltpu.VMEM((SBK,), jnp.int32),
            pltpu.VMEM((SBK,), jnp.int32),
            pltpu.VMEM((QCAP,), jnp.int32),
            pltpu.VMEM((2 * PB,), jnp.int32),
            pltpu.VMEM((2 * PB,), jnp.int32),
            pltpu.VMEM((2 * PB, 128), jnp.float32),
            pltpu.VMEM_SHARED((AR + 8, 128), jnp.float32),
            pltpu.SemaphoreType.DMA,
            pltpu.SemaphoreType.DMA,
        ],
        compiler_params=_NOLAYOUT,
        **_MESH,
    )


def _make_pool():
    rows = NPAD // NW
    BRK = 56

    def body(h_hbm, b_hbm, out_hbm, hbuf, bbuf, macc, sem):
        c = lax.axis_index("c")
        s = lax.axis_index("s")
        wid = c * NS + s
        base = wid * rows

        def im(r, _):
            for j in range(16):
                macc[r, pl.ds(j * 16, 16)] = jnp.full((16,), -jnp.inf, jnp.float32)
            return 0

        lax.fori_loop(0, G, im, 0)
        pltpu.sync_copy(b_hbm.at[pl.ds(base, rows)], bbuf.at[pl.ds(0, rows)])

        def blk(k, _):
            pltpu.sync_copy(h_hbm.at[pl.ds(base + k * BRK, BRK), :], hbuf)

            def row(i, _):
                gr = base + k * BRK + i

                @pl.when(gr < N)
                def _():
                    g = bbuf[pl.ds(k * BRK + i, 16)][0]
                    for j in range(16):
                        a = macc[g, pl.ds(j * 16, 16)]
                        hv = hbuf[i, pl.ds(j * 16, 16)]
                        macc[g, pl.ds(j * 16, 16)] = jnp.maximum(a, hv)

                return 0

            lax.fori_loop(0, BRK, row, 0)
            return 0

        lax.fori_loop(0, rows // BRK, blk, 0)
        pltpu.sync_copy(macc, out_hbm.at[wid])

    return pl.kernel(
        body,
        out_type=jax.ShapeDtypeStruct((NW, G, 256), jnp.float32),
        scratch_types=[
            pltpu.VMEM((BRK, 256), jnp.float32),
            pltpu.VMEM((rows + 16,), jnp.int32),
            pltpu.VMEM((G, 256), jnp.float32),
            pltpu.SemaphoreType.DMA,
        ],
        **_MESH,
    )


def _prep_body(deg0_ref, deg1_ref, x_ref, w_ref, t1_ref, dinv_ref):
    deg = deg0_ref[...] + deg1_ref[...] + 1.0
    d = lax.rsqrt(deg)
    dinv_ref[...] = d
    t1_ref[...] = jnp.dot(x_ref[...], w_ref[...],
                          preferred_element_type=jnp.float32) * d


def _prep(deg0, deg1, xpad, W1p):
    return pl.pallas_call(
        _prep_body,
        grid=(NRB,),
        in_specs=[
            pl.BlockSpec((RB, 1), lambda i: (i, 0)),
            pl.BlockSpec((RB, 1), lambda i: (i, 0)),
            pl.BlockSpec((RB, 16), lambda i: (i, 0)),
            pl.BlockSpec((16, 128), lambda i: (0, 0)),
        ],
        out_specs=[
            pl.BlockSpec((RB, 128), lambda i: (i, 0)),
            pl.BlockSpec((RB, 1), lambda i: (i, 0)),
        ],
        out_shape=[
            jax.ShapeDtypeStruct((NPAD, 128), jnp.float32),
            jax.ShapeDtypeStruct((NPAD, 1), jnp.float32),
        ],
    )(deg0, deg1, xpad, W1p)


def _elem_body(y_ref, t_ref, dinv_ref, b_ref, out_ref):
    d = dinv_ref[...]
    h = jnp.maximum((y_ref[0] + y_ref[1] + t_ref[...]) * d + b_ref[...], 0.0)
    out_ref[...] = h * d


def _elem(y, t, dinv, b):
    w = t.shape[1]
    return pl.pallas_call(
        _elem_body,
        grid=(NRB,),
        in_specs=[
            pl.BlockSpec((2, RB, w), lambda i: (0, i, 0)),
            pl.BlockSpec((RB, w), lambda i: (i, 0)),
            pl.BlockSpec((RB, 1), lambda i: (i, 0)),
            pl.BlockSpec((1, w), lambda i: (0, 0)),
        ],
        out_specs=pl.BlockSpec((RB, w), lambda i: (i, 0)),
        out_shape=jax.ShapeDtypeStruct((NPAD, w), jnp.float32),
    )(y, t, dinv, b)


def _layer_body(y_ref, t_ref, dinv_ref, w_ref, b_ref, out_ref, *, relu, scale_out):
    d = dinv_ref[...]
    z = (y_ref[0] + y_ref[1] + t_ref[...]) * d
    o = jnp.dot(z, w_ref[...], preferred_element_type=jnp.float32) + b_ref[...]
    if relu:
        o = jnp.maximum(o, 0.0)
    if scale_out:
        o = o * d
    out_ref[...] = o


def _layer(y, t, dinv, w, b, *, relu, scale_out):
    wi, wo = w.shape
    return pl.pallas_call(
        functools.partial(_layer_body, relu=relu, scale_out=scale_out),
        grid=(NRB,),
        in_specs=[
            pl.BlockSpec((2, RB, wi), lambda i: (0, i, 0)),
            pl.BlockSpec((RB, wi), lambda i: (i, 0)),
            pl.BlockSpec((RB, 1), lambda i: (i, 0)),
            pl.BlockSpec((wi, wo), lambda i: (0, 0)),
            pl.BlockSpec((1, wo), lambda i: (0, 0)),
        ],
        out_specs=pl.BlockSpec((RB, wo), lambda i: (i, 0)),
        out_shape=jax.ShapeDtypeStruct((NPAD, wo), jnp.float32),
    )(y, t, dinv, w, b)


def _head_body(p_ref, w2_ref, b2_ref, w3_ref, b3_ref, wl_ref, bl_ref, out_ref):
    hp = jnp.max(p_ref[...], axis=0)
    h = jax.nn.relu(jnp.dot(hp, w2_ref[...], preferred_element_type=jnp.float32) + b2_ref[...])
    h = jax.nn.relu(jnp.dot(h, w3_ref[...], preferred_element_type=jnp.float32) + b3_ref[...])
    out_ref[...] = jnp.dot(h, wl_ref[...], preferred_element_type=jnp.float32) + bl_ref[...]


def _head(pool, Wl2, bl2, Wl3, bl3, Wlin, blin):
    return pl.pallas_call(
        _head_body,
        out_shape=jax.ShapeDtypeStruct((G, Wlin.shape[1]), jnp.float32),
    )(pool, Wl2, bl2[None, :], Wl3, bl3[None, :], Wlin, blin[None, :])


def kernel(x, edge_index, batch, W1, b1, W2, b2, W3, b3, W4, b4, Wl2, bl2, Wl3, bl3, Wlin, blin):
    E = edge_index.shape[1]
    ept = ((E + NW * SBK - 1) // (NW * SBK)) * SBK
    epad = NW * ept
    src_p = jnp.concatenate([edge_index[0], jnp.zeros((epad - E,), jnp.int32)])
    dst_p = jnp.concatenate([edge_index[1], jnp.full((epad - E,), -1, jnp.int32)])
    xpad = jnp.pad(x, ((0, NPAD - N), (0, 16 - x.shape[1])))
    batch_p = jnp.pad(batch, (0, NPAD - N))
    W1p = jnp.pad(W1, ((0, 16 - W1.shape[0]), (0, 0)))

    degp = _make_deg(ept)(dst_p)
    degc = degp.reshape(NC, NPAD, 1)
    t1, dinv = _prep(degc[0], degc[1], xpad, W1p)

    agg128 = _make_agg(7, ept, 1, 128)
    agg256 = _make_agg(14, ept, 2, 64)

    def agg_w(agg, t, expand):
        y = agg(src_p, dst_p, t.reshape(NPAD * expand, 128))
        return y.reshape(NC, NPAD, 128 * expand)

    a1 = agg_w(agg128, t1, 1)
    t2 = _elem(a1, t1, dinv, b1[None, :])
    a2 = agg_w(agg128, t2, 1)
    t3 = _layer(a2, t2, dinv, W2, b2[None, :], relu=True, scale_out=True)
    a3 = agg_w(agg256, t3, 2)
    t4 = _layer(a3, t3, dinv, W3, b3[None, :], relu=True, scale_out=True)
    a4 = agg_w(agg256, t4, 2)
    h4 = _layer(a4, t4, dinv, W4, b4[None, :], relu=False, scale_out=False)

    pool = _make_pool()(h4, batch_p)
    out = _head(pool, Wl2, bl2, Wl3, bl3, Wlin, blin)
    return jnp.squeeze(out)


# speedup vs baseline: 9.1149x; 1.0159x over previous
import functools

import jax
import jax.numpy as jnp
from jax import lax
from jax.experimental import pallas as pl
from jax.experimental.pallas import tpu as pltpu
from jax.experimental.pallas import tpu_sc as plsc

N = 50000
G = 256
NC = 2
NS = 16
NW = NC * NS
NPAD = 50176
RB = 512
NRB = NPAD // RB
SBK = 3584

_MESH = dict(mesh=plsc.VectorSubcoreMesh(core_axis_name="c", subcore_axis_name="s"))
_NOLAYOUT = pltpu.CompilerParams(needs_layout_passes=False)


def _make_deg(ept):
    nblk = ept // SBK
    sl = NPAD // NS

    def body(dst_hbm, deg_hbm, dst_st, didx, dval, zbuf, acc, sem):
        c = lax.axis_index("c")
        s = lax.axis_index("s")
        wid = c * NS + s

        def z(i, _):
            zbuf[pl.ds(i * 16, 16)] = jnp.zeros((16,), jnp.float32)
            return 0

        lax.fori_loop(0, sl // 16, z, 0)
        pltpu.sync_copy(zbuf, acc.at[pl.ds(s * sl, sl)])
        plsc.subcore_barrier()

        def blk(h, _):
            base = wid * ept + h * SBK
            pltpu.sync_copy(dst_hbm.at[pl.ds(base, SBK)], dst_st)

            def chunk(j, _):
                for t in range(8):
                    dv = dst_st[pl.ds(j * 128 + t * 16, 16)]
                    m = dv >= 0
                    didx[pl.ds(t * 16, 16)] = jnp.where(m, dv, 0)
                    dval[pl.ds(t * 16, 16)] = jnp.where(m, 1.0, 0.0)
                pltpu.sync_copy(dval, acc.at[didx], add=True)
                return 0

            lax.fori_loop(0, SBK // 128, chunk, 0)
            return 0

        lax.fori_loop(0, nblk, blk, 0)
        plsc.subcore_barrier()
        pltpu.sync_copy(acc.at[pl.ds(s * sl, sl)], zbuf)
        pltpu.sync_copy(zbuf, deg_hbm.at[pl.ds(c * NPAD + s * sl, sl)])

    return pl.kernel(
        body,
        out_type=jax.ShapeDtypeStruct((NC * NPAD,), jnp.float32),
        scratch_types=[
            pltpu.VMEM((SBK,), jnp.int32),
            pltpu.VMEM((128,), jnp.int32),
            pltpu.VMEM((128,), jnp.float32),
            pltpu.VMEM((sl,), jnp.float32),
            pltpu.VMEM_SHARED((NPAD,), jnp.float32),
            pltpu.SemaphoreType.DMA,
        ],
        **_MESH,
    )


def _make_agg(npass, ept, expand, B):
    C = NPAD // npass
    NV = B // 16
    PB = B * expand
    AR = expand * C
    nblk = ept // SBK
    zr = AR // NS
    WBK = 8
    QCAP = ept + B

    def body(src_hbm, dst_hbm, u_hbm, y_hbm,
             src_st, dst_st, sel, gidx, gdl, gbuf, acc,
             semA, semB, semSA, semSB):
        c = lax.axis_index("c")
        s = lax.axis_index("s")
        wid = c * NS + s
        lane = lax.iota(jnp.int32, 16)

        def views(slot):
            gi = gidx.at[pl.ds(slot * PB, PB)]
            gb = gbuf.at[pl.ds(slot * PB, PB), :]
            sem = semA if slot == 0 else semB
            return gi, gb, sem

        def sviews(slot):
            gd = gdl.at[pl.ds(slot * PB, PB)]
            gb = gbuf.at[pl.ds(slot * PB, PB), :]
            sem = semSA if slot == 0 else semSB
            return gd, gb, sem

        def build_fire(f, slot):
            for rr in range(NV):
                v = sel[pl.ds(f * B + rr * 16, 16)]
                sv = jnp.bitwise_and(v, 0xFFFF)
                dl = lax.shift_right_logical(v, 16)
                if expand == 1:
                    gidx[pl.ds(slot * PB + rr * 16, 16)] = sv
                    gdl[pl.ds(slot * PB + rr * 16, 16)] = dl
                else:
                    sv2 = sv * 2
                    dl2 = dl * 2
                    pos = slot * PB + rr * 32 + lane * 2
                    plsc.store_scatter(gidx, [pos], sv2)
                    plsc.store_scatter(gidx, [pos + 1], sv2 + 1)
                    plsc.store_scatter(gdl, [pos], dl2)
                    plsc.store_scatter(gdl, [pos + 1], dl2 + 1)
            gi, gb, sem = views(slot)
            pltpu.async_copy(u_hbm.at[gi], gb, sem)

        def retire(slot):
            gi, gb, sem = views(slot)
            pltpu.make_async_copy(u_hbm.at[gi], gb, sem).wait()
            gd, gb, ssem = sviews(slot)
            pltpu.async_copy(gb, acc.at[gd], ssem, add=True)

        def wait_scat(slot):
            gd, gb, ssem = sviews(slot)
            pltpu.make_async_copy(gb, acc.at[gd], ssem).wait()

        def step(f, _):
            odd = lax.rem(f, 2) == 1
            even = jnp.logical_not(odd)

            @pl.when((f >= 2) & odd)
            def _():
                wait_scat(1)

            @pl.when((f >= 2) & even)
            def _():
                wait_scat(0)

            @pl.when(odd)
            def _():
                build_fire(f, 1)

            @pl.when(even)
            def _():
                build_fire(f, 0)

            @pl.when((f >= 1) & odd)
            def _():
                retire(0)

            @pl.when((f >= 1) & even)
            def _():
                retire(1)

            return 0

        def one_pass(p, _):
            lo = p * C

            def zrow(i, _):
                for j in range(8):
                    gbuf[i, pl.ds(j * 16, 16)] = jnp.zeros((16,), jnp.float32)
                return 0

            lax.fori_loop(0, WBK, zrow, 0)

            def zacc(k, _):
                pltpu.sync_copy(gbuf.at[pl.ds(0, WBK), :],
                                acc.at[pl.ds(s * zr + k * WBK, WBK), :])
                return 0

            lax.fori_loop(0, zr // WBK, zacc, 0)
            plsc.subcore_barrier()

            def blk(h, carry):
                qv, nfdone = carry
                base = wid * ept + h * SBK
                pltpu.sync_copy(src_hbm.at[pl.ds(base, SBK)], src_st)
                pltpu.sync_copy(dst_hbm.at[pl.ds(base, SBK)], dst_st)

                def vrg(i, q):
                    sv = src_st[pl.ds(i * 16, 16)]
                    dv = dst_st[pl.ds(i * 16, 16)]
                    m = (dv >= lo) & (dv < lo + C)
                    mi = jnp.where(m, 1, 0)
                    pos = q + plsc.cumsum(mi) - mi
                    packed = jnp.bitwise_or(jnp.bitwise_and(sv, 0xFFFF),
                                            jnp.left_shift(dv - lo, 16))
                    plsc.store_scatter(sel, [pos], packed, mask=m)
                    return q + plsc.all_reduce_population_count(m)

                qv = lax.fori_loop(0, SBK // 16, vrg, qv)
                nfull = qv[0] // B
                lax.fori_loop(nfdone, nfull, step, 0)
                return qv, nfull

            qv, nfull = lax.fori_loop(0, nblk, blk,
                                      (jnp.zeros((16,), jnp.int32), jnp.int32(0)))

            qrem = qv[0] - nfull * B

            @pl.when(qrem > 0)
            def _():
                dmy = jnp.full((16,), C << 16, jnp.int32)
                for t in range(NV):
                    pos = nfull * B + t * 16 + lane
                    plsc.store_scatter(sel, [pos], dmy, mask=pos >= qv)
                step(nfull, 0)

            total = nfull + jnp.where(qrem > 0, 1, 0)
            last_odd = lax.rem(total - 1, 2) == 1
            last_even = jnp.logical_not(last_odd)

            @pl.when((total > 0) & last_odd)
            def _():
                retire(1)
                wait_scat(1)

            @pl.when((total > 0) & last_even)
            def _():
                retire(0)
                wait_scat(0)

            @pl.when((total >= 2) & last_odd)
            def _():
                wait_scat(0)

            @pl.when((total >= 2) & last_even)
            def _():
                wait_scat(1)

            plsc.subcore_barrier()

            def wb(k, _):
                r = s * zr + k * WBK
                pltpu.sync_copy(acc.at[pl.ds(r, WBK), :], gbuf.at[pl.ds(0, WBK), :])
                pltpu.sync_copy(gbuf.at[pl.ds(0, WBK), :],
                                y_hbm.at[c, pl.ds(p * AR + r, WBK), :])
                return 0

            lax.fori_loop(0, zr // WBK, wb, 0)
            plsc.subcore_barrier()
            return 0

        lax.fori_loop(0, npass, one_pass, 0)

    return pl.kernel(
        body,
        out_type=jax.ShapeDtypeStruct((NC, NPAD * expand, 128), jnp.float32),
        scratch_types=[
            pltpu.VMEM((SBK,), jnp.int32),
            pltpu.VMEM((SBK,), jnp.int32),
            pltpu.VMEM((QCAP,), jnp.int32),
            pltpu.VMEM((2 * PB,), jnp.int32),
            pltpu.VMEM((2 * PB,), jnp.int32),
            pltpu.VMEM((2 * PB, 128), jnp.float32),
            pltpu.VMEM_SHARED((AR + 8, 128), jnp.float32),
            pltpu.SemaphoreType.DMA,
            pltpu.SemaphoreType.DMA,
            pltpu.SemaphoreType.DMA,
            pltpu.SemaphoreType.DMA,
        ],
        compiler_params=_NOLAYOUT,
        **_MESH,
    )


def _make_pool():
    rows = NPAD // NW
    BRK = 56

    def body(h_hbm, b_hbm, out_hbm, hbuf, bbuf, macc, sem):
        c = lax.axis_index("c")
        s = lax.axis_index("s")
        wid = c * NS + s
        base = wid * rows

        def im(r, _):
            for j in range(16):
                macc[r, pl.ds(j * 16, 16)] = jnp.full((16,), -jnp.inf, jnp.float32)
            return 0

        lax.fori_loop(0, G, im, 0)
        pltpu.sync_copy(b_hbm.at[pl.ds(base, rows)], bbuf.at[pl.ds(0, rows)])

        def blk(k, _):
            pltpu.sync_copy(h_hbm.at[pl.ds(base + k * BRK, BRK), :], hbuf)

            def row(i, _):
                gr = base + k * BRK + i

                @pl.when(gr < N)
                def _():
                    g = bbuf[pl.ds(k * BRK + i, 16)][0]
                    for j in range(16):
                        a = macc[g, pl.ds(j * 16, 16)]
                        hv = hbuf[i, pl.ds(j * 16, 16)]
                        macc[g, pl.ds(j * 16, 16)] = jnp.maximum(a, hv)

                return 0

            lax.fori_loop(0, BRK, row, 0)
            return 0

        lax.fori_loop(0, rows // BRK, blk, 0)
        pltpu.sync_copy(macc, out_hbm.at[wid])

    return pl.kernel(
        body,
        out_type=jax.ShapeDtypeStruct((NW, G, 256), jnp.float32),
        scratch_types=[
            pltpu.VMEM((BRK, 256), jnp.float32),
            pltpu.VMEM((rows + 16,), jnp.int32),
            pltpu.VMEM((G, 256), jnp.float32),
            pltpu.SemaphoreType.DMA,
        ],
        **_MESH,
    )


def _prep_body(deg0_ref, deg1_ref, x_ref, w_ref, t1_ref, dinv_ref):
    deg = deg0_ref[...] + deg1_ref[...] + 1.0
    d = lax.rsqrt(deg)
    dinv_ref[...] = d
    t1_ref[...] = jnp.dot(x_ref[...], w_ref[...],
                          preferred_element_type=jnp.float32) * d


def _prep(deg0, deg1, xpad, W1p):
    return pl.pallas_call(
        _prep_body,
        grid=(NRB,),
        in_specs=[
            pl.BlockSpec((RB, 1), lambda i: (i, 0)),
            pl.BlockSpec((RB, 1), lambda i: (i, 0)),
            pl.BlockSpec((RB, 16), lambda i: (i, 0)),
            pl.BlockSpec((16, 128), lambda i: (0, 0)),
        ],
        out_specs=[
            pl.BlockSpec((RB, 128), lambda i: (i, 0)),
            pl.BlockSpec((RB, 1), lambda i: (i, 0)),
        ],
        out_shape=[
            jax.ShapeDtypeStruct((NPAD, 128), jnp.float32),
            jax.ShapeDtypeStruct((NPAD, 1), jnp.float32),
        ],
    )(deg0, deg1, xpad, W1p)


def _elem_body(y_ref, t_ref, dinv_ref, b_ref, out_ref):
    d = dinv_ref[...]
    h = jnp.maximum((y_ref[0] + y_ref[1] + t_ref[...]) * d + b_ref[...], 0.0)
    out_ref[...] = h * d


def _elem(y, t, dinv, b):
    w = t.shape[1]
    return pl.pallas_call(
        _elem_body,
        grid=(NRB,),
        in_specs=[
            pl.BlockSpec((2, RB, w), lambda i: (0, i, 0)),
            pl.BlockSpec((RB, w), lambda i: (i, 0)),
            pl.BlockSpec((RB, 1), lambda i: (i, 0)),
            pl.BlockSpec((1, w), lambda i: (0, 0)),
        ],
        out_specs=pl.BlockSpec((RB, w), lambda i: (i, 0)),
        out_shape=jax.ShapeDtypeStruct((NPAD, w), jnp.float32),
    )(y, t, dinv, b)


def _layer_body(y_ref, t_ref, dinv_ref, w_ref, b_ref, out_ref, *, relu, scale_out):
    d = dinv_ref[...]
    z = (y_ref[0] + y_ref[1] + t_ref[...]) * d
    o = jnp.dot(z, w_ref[...], preferred_element_type=jnp.float32) + b_ref[...]
    if relu:
        o = jnp.maximum(o, 0.0)
    if scale_out:
        o = o * d
    out_ref[...] = o


def _layer(y, t, dinv, w, b, *, relu, scale_out):
    wi, wo = w.shape
    return pl.pallas_call(
        functools.partial(_layer_body, relu=relu, scale_out=scale_out),
        grid=(NRB,),
        in_specs=[
            pl.BlockSpec((2, RB, wi), lambda i: (0, i, 0)),
            pl.BlockSpec((RB, wi), lambda i: (i, 0)),
            pl.BlockSpec((RB, 1), lambda i: (i, 0)),
            pl.BlockSpec((wi, wo), lambda i: (0, 0)),
            pl.BlockSpec((1, wo), lambda i: (0, 0)),
        ],
        out_specs=pl.BlockSpec((RB, wo), lambda i: (i, 0)),
        out_shape=jax.ShapeDtypeStruct((NPAD, wo), jnp.float32),
    )(y, t, dinv, w, b)


def _head_body(p_ref, w2_ref, b2_ref, w3_ref, b3_ref, wl_ref, bl_ref, out_ref):
    hp = jnp.max(p_ref[...], axis=0)
    h = jax.nn.relu(jnp.dot(hp, w2_ref[...], preferred_element_type=jnp.float32) + b2_ref[...])
    h = jax.nn.relu(jnp.dot(h, w3_ref[...], preferred_element_type=jnp.float32) + b3_ref[...])
    out_ref[...] = jnp.dot(h, wl_ref[...], preferred_element_type=jnp.float32) + bl_ref[...]


def _head(pool, Wl2, bl2, Wl3, bl3, Wlin, blin):
    return pl.pallas_call(
        _head_body,
        out_shape=jax.ShapeDtypeStruct((G, Wlin.shape[1]), jnp.float32),
    )(pool, Wl2, bl2[None, :], Wl3, bl3[None, :], Wlin, blin[None, :])


def kernel(x, edge_index, batch, W1, b1, W2, b2, W3, b3, W4, b4, Wl2, bl2, Wl3, bl3, Wlin, blin):
    E = edge_index.shape[1]
    ept = ((E + NW * SBK - 1) // (NW * SBK)) * SBK
    epad = NW * ept
    src_p = jnp.concatenate([edge_index[0], jnp.zeros((epad - E,), jnp.int32)])
    dst_p = jnp.concatenate([edge_index[1], jnp.full((epad - E,), -1, jnp.int32)])
    xpad = jnp.pad(x, ((0, NPAD - N), (0, 16 - x.shape[1])))
    batch_p = jnp.pad(batch, (0, NPAD - N))
    W1p = jnp.pad(W1, ((0, 16 - W1.shape[0]), (0, 0)))

    degp = _make_deg(ept)(dst_p)
    degc = degp.reshape(NC, NPAD, 1)
    t1, dinv = _prep(degc[0], degc[1], xpad, W1p)

    agg128 = _make_agg(7, ept, 1, 128)
    agg256 = _make_agg(14, ept, 2, 64)

    def agg_w(agg, t, expand):
        y = agg(src_p, dst_p, t.reshape(NPAD * expand, 128))
        return y.reshape(NC, NPAD, 128 * expand)

    a1 = agg_w(agg128, t1, 1)
    t2 = _elem(a1, t1, dinv, b1[None, :])
    a2 = agg_w(agg128, t2, 1)
    t3 = _layer(a2, t2, dinv, W2, b2[None, :], relu=True, scale_out=True)
    a3 = agg_w(agg256, t3, 2)
    t4 = _layer(a3, t3, dinv, W3, b3[None, :], relu=True, scale_out=True)
    a4 = agg_w(agg256, t4, 2)
    h4 = _layer(a4, t4, dinv, W4, b4[None, :], relu=False, scale_out=False)

    pool = _make_pool()(h4, batch_p)
    out = _head(pool, Wl2, bl2, Wl3, bl3, Wlin, blin)
    return jnp.squeeze(out)
